# baseline (speedup 1.0000x reference)
import jax
import jax.numpy as jnp
from jax.experimental import pallas as pl
from jax.experimental.pallas import tpu as pltpu

N = 1024
NB = 1024
EMBED = 128
RNN = 256
B = 32
T = 50
ALPHA = 0.5

_RBC = 128
_NIC = N // _RBC
_RB = 512
_NI = N // _RB
_BB = 8
_XB = _BB * T
_NJ = B // _BB
_CS = _NIC
_GS = _CS + 4 * _NI
_LS = _GS + _NJ


def _body(F_ref, A_ref, x_ref, ib_ref, linW_ref, linb_ref, projW_ref,
          projb_ref, wih_ref, whh_ref, bih_ref, bhh_ref, h0_ref, c0_ref,
          sl_ref, h2i_ref, out_ref,
          ha_s, hb_s, hb2_s, h1z_s, h2a_s, h2b_s, xs_s, xg_s, deg_s, dinv_s):
    s = pl.program_id(0)

    @pl.when(s < _CS)
    def _():
        ablk = [A_ref[e] for e in range(4)]
        for w, dst in ((0, ha_s), (1, hb_s), (2, hb2_s)):
            for cc in range(2):
                acc = F_ref[w, cc, 0] * ablk[0]
                for e in range(1, 4):
                    acc = acc + F_ref[w, cc, e] * ablk[e]
                dst[pl.ds(cc * N + s * _RBC, _RBC), :] = (
                    acc.astype(jnp.bfloat16))

    sp = s - _CS
    c = sp // (2 * _NI)
    p = (sp // _NI) % 2
    i = sp % _NI
    rows = pl.ds(i * _RB, _RB)
    crows = pl.ds(c * N + i * _RB, _RB)
    call = pl.ds(c * N, N)
    graph = jnp.logical_and(s >= _CS, s < _GS)

    @pl.when(jnp.logical_and(graph, jnp.logical_and(p == 0, i == 0)))
    def _():
        deg_s[...] = jnp.zeros_like(deg_s)

    @pl.when(jnp.logical_and(graph, p == 0))
    def _():
        h1 = jnp.dot(ha_s[crows, :], hb_s[call, :],
                     preferred_element_type=jnp.float32)
        col = jax.lax.broadcasted_iota(jnp.int32, (_RB, N), 1)
        row = jax.lax.broadcasted_iota(jnp.int32, (_RB, N), 0) + i * _RB
        h1 = jnp.where(col == row, 0.0, h1)
        deg_s[...] += jnp.sum(h1, axis=0, keepdims=True)
        h1z_s[rows, :] = h1.astype(jnp.bfloat16)

    @pl.when(jnp.logical_and(graph, jnp.logical_and(p == 1, i == 0)))
    def _():
        deg = deg_s[...]
        dinv_s[...] = jnp.where(deg > 0, 1.0 / deg, 0.0)

    @pl.when(jnp.logical_and(graph, p == 1))
    def _():
        h1z = (h1z_s[rows, :] * dinv_s[...]).astype(jnp.bfloat16)
        h2blk = jnp.dot(h1z, hb2_s[call, :],
                        preferred_element_type=jnp.float32
                        ).astype(jnp.bfloat16)

        @pl.when(c == 0)
        def _():
            h2a_s[rows, :] = h2blk

        @pl.when(c == 1)
        def _():
            h2b_s[rows, :] = h2blk

    j = s - _GS

    @pl.when(jnp.logical_and(s >= _GS, s < _LS))
    def _():
        one = jnp.bfloat16(0.0)
        xb16 = x_ref[...].astype(jnp.bfloat16)
        ib = jnp.maximum(ib_ref[...], 0.0).astype(jnp.bfloat16)
        xd = xb16 * ib
        t0 = jnp.dot(xb16, h2a_s[...],
                     preferred_element_type=jnp.float32).astype(jnp.bfloat16)
        e0 = jnp.maximum(xd, one) + jnp.maximum(t0, one)
        t1 = jnp.dot(xb16, h2b_s[...],
                     preferred_element_type=jnp.float32).astype(jnp.bfloat16)
        e1 = jnp.maximum(xd + jnp.maximum(t1, one), one)
        dn = (((1,), (1,)), ((), ()))
        eb0 = (jax.lax.dot_general(
            e0, linW_ref[0].astype(jnp.bfloat16), dn,
            preferred_element_type=jnp.float32)
            + linb_ref[0]).astype(jnp.bfloat16)
        eb1 = (jax.lax.dot_general(
            e1, linW_ref[1].astype(jnp.bfloat16), dn,
            preferred_element_type=jnp.float32)
            + linb_ref[1]).astype(jnp.bfloat16)
        comb = (jax.lax.dot_general(
                    eb0, projW_ref[:, 0:EMBED].astype(jnp.bfloat16), dn,
                    preferred_element_type=jnp.float32)
                + jax.lax.dot_general(
                    eb1, projW_ref[:, EMBED:2 * EMBED].astype(jnp.bfloat16),
                    dn, preferred_element_type=jnp.float32)
                + projb_ref[...]).astype(jnp.bfloat16)
        for jj in range(_NJ):
            @pl.when(j == jj)
            def _():
                for b in range(_BB):
                    xs_s[:, jj * _BB + b, :] = comb[b * T:(b + 1) * T, :]

    @pl.when(s == _LS)
    def _():
        dn = (((1,), (1,)), ((), ()))
        xs_flat = xs_s[...].reshape(T * B, EMBED)
        xg_s[...] = (jax.lax.dot_general(xs_flat,
                                         wih_ref[...].astype(jnp.bfloat16),
                                         dn,
                                         preferred_element_type=jnp.float32)
                     + bih_ref[...] + bhh_ref[...])
        tgt = sl_ref[...] - 1

        def step(t, carry):
            h, cc, acc = carry
            xt = xg_s[pl.ds(t * B, B), :]
            gates = xt + jax.lax.dot_general(
                h, whh_ref[...], dn, preferred_element_type=jnp.float32)
            i_ = jax.nn.sigmoid(gates[:, 0:RNN])
            f_ = jax.nn.sigmoid(gates[:, RNN:2 * RNN])
            g_ = jnp.tanh(gates[:, 2 * RNN:3 * RNN])
            o_ = jax.nn.sigmoid(gates[:, 3 * RNN:4 * RNN])
            cc = f_ * cc + i_ * g_
            h = o_ * jnp.tanh(cc)
            acc = jnp.where(tgt == t, h, acc)
            return h, cc, acc

        h0 = h0_ref[...]
        _, _, actual = jax.lax.fori_loop(
            0, T, step, (h0, c0_ref[...], jnp.zeros_like(h0)))
        scores = jax.lax.dot_general(actual, h2i_ref[...], dn,
                                     preferred_element_type=jnp.float32)
        probs = jax.nn.sigmoid(scores)
        ib = jnp.maximum(ib_ref[...], 0.0)
        out_ref[...] = (1.0 - ALPHA) * probs + ALPHA * (probs * ib)


def kernel(A, seqs, seq_len, h0, c0, Wc1, Wc2, Wc3, I_B, lin_W, lin_b,
           proj_W, proj_b, W_ih, W_hh, b_ih, b_hh, h2i_W):
    F = jax.nn.softmax(jnp.stack([Wc1, Wc2, Wc3]), axis=2)
    x = seqs.reshape(B * T, NB)
    ib2 = I_B.reshape(1, NB)

    out = pl.pallas_call(
        _body,
        grid=(_LS + 1,),
        in_specs=[
            pl.BlockSpec(memory_space=pltpu.SMEM),
            pl.BlockSpec((4, _RBC, N), lambda s: (0, jnp.clip(s, 0, _NIC - 1),
                                                  0)),
            pl.BlockSpec((_XB, NB), lambda s: (jnp.clip(s - _GS, 0, _NJ - 1),
                                               0)),
            pl.BlockSpec((1, NB), lambda s: (0, 0)),
            pl.BlockSpec((2, EMBED, NB), lambda s: (0, 0, 0)),
            pl.BlockSpec((2, 1, EMBED), lambda s: (0, 0, 0)),
            pl.BlockSpec((EMBED, 2 * EMBED), lambda s: (0, 0)),
            pl.BlockSpec((1, EMBED), lambda s: (0, 0)),
            pl.BlockSpec((4 * RNN, EMBED), lambda s: (0, 0)),
            pl.BlockSpec((4 * RNN, RNN), lambda s: (0, 0)),
            pl.BlockSpec((1, 4 * RNN), lambda s: (0, 0)),
            pl.BlockSpec((1, 4 * RNN), lambda s: (0, 0)),
            pl.BlockSpec((B, RNN), lambda s: (0, 0)),
            pl.BlockSpec((B, RNN), lambda s: (0, 0)),
            pl.BlockSpec((B, 1), lambda s: (0, 0)),
            pl.BlockSpec((NB, RNN), lambda s: (0, 0)),
        ],
        out_specs=pl.BlockSpec((B, NB), lambda s: (0, 0)),
        out_shape=jax.ShapeDtypeStruct((B, NB), jnp.float32),
        scratch_shapes=[
            pltpu.VMEM((2 * N, N), jnp.bfloat16),
            pltpu.VMEM((2 * N, N), jnp.bfloat16),
            pltpu.VMEM((2 * N, N), jnp.bfloat16),
            pltpu.VMEM((N, N), jnp.bfloat16),
            pltpu.VMEM((N, N), jnp.bfloat16),
            pltpu.VMEM((N, N), jnp.bfloat16),
            pltpu.VMEM((T, B, EMBED), jnp.bfloat16),
            pltpu.VMEM((T * B, 4 * RNN), jnp.float32),
            pltpu.VMEM((1, N), jnp.float32),
            pltpu.VMEM((1, N), jnp.float32),
        ],
        compiler_params=pltpu.CompilerParams(
            dimension_semantics=("arbitrary",)),
    )(F, A, x, ib2, lin_W, lin_b.reshape(2, 1, EMBED), proj_W,
      proj_b.reshape(1, EMBED), W_ih, W_hh, b_ih.reshape(1, 4 * RNN),
      b_hh.reshape(1, 4 * RNN), h0.reshape(B, RNN), c0.reshape(B, RNN),
      seq_len.astype(jnp.int32).reshape(B, 1), h2i_W)

    return out


# speedup vs baseline: 1.0323x; 1.0323x over previous
import jax
import jax.numpy as jnp
from jax.experimental import pallas as pl
from jax.experimental.pallas import tpu as pltpu

N = 1024
NB = 1024
EMBED = 128
RNN = 256
B = 32
T = 50
ALPHA = 0.5

_RBC = 512
_NIC = N // _RBC
_RB = 1024
_NI = N // _RB
_BB = 8
_XB = _BB * T
_NJ = B // _BB
_CS = _NIC
_GS = _CS + 4 * _NI
_LS = _GS + _NJ


def _body(F_ref, A_ref, x_ref, ib_ref, linW_ref, linb_ref, projW_ref,
          projb_ref, wih_ref, whh_ref, bih_ref, bhh_ref, h0_ref, c0_ref,
          sl_ref, h2i_ref, out_ref,
          ha_s, hb_s, hb2_s, h1z_s, h2ab_s, xs_s, xg_s, deg_s, dinv_s):
    s = pl.program_id(0)

    @pl.when(s < _CS)
    def _():
        ablk = [A_ref[e] for e in range(4)]
        for w, dst in ((0, ha_s), (1, hb_s), (2, hb2_s)):
            for cc in range(2):
                acc = F_ref[w, cc, 0] * ablk[0]
                for e in range(1, 4):
                    acc = acc + F_ref[w, cc, e] * ablk[e]
                dst[pl.ds(cc * N + s * _RBC, _RBC), :] = (
                    acc.astype(jnp.bfloat16))

    sp = s - _CS
    c = sp // (2 * _NI)
    p = (sp // _NI) % 2
    i = sp % _NI
    rows = pl.ds(i * _RB, _RB)
    crows = pl.ds(c * N + i * _RB, _RB)
    call = pl.ds(c * N, N)
    graph = jnp.logical_and(s >= _CS, s < _GS)

    @pl.when(jnp.logical_and(graph, jnp.logical_and(p == 0, i == 0)))
    def _():
        deg_s[...] = jnp.zeros_like(deg_s)

    @pl.when(jnp.logical_and(graph, p == 0))
    def _():
        h1 = jnp.dot(ha_s[crows, :], hb_s[call, :],
                     preferred_element_type=jnp.float32)
        col = jax.lax.broadcasted_iota(jnp.int32, (_RB, N), 1)
        row = jax.lax.broadcasted_iota(jnp.int32, (_RB, N), 0) + i * _RB
        h1 = jnp.where(col == row, 0.0, h1)
        deg_s[...] += jnp.sum(h1, axis=0, keepdims=True)
        h1z_s[rows, :] = h1.astype(jnp.bfloat16)

    @pl.when(jnp.logical_and(graph, jnp.logical_and(p == 1, i == 0)))
    def _():
        deg = deg_s[...]
        dinv_s[...] = jnp.where(deg > 0, 1.0 / deg, 0.0)

    @pl.when(jnp.logical_and(graph, p == 1))
    def _():
        h1z = (h1z_s[rows, :] * dinv_s[...]).astype(jnp.bfloat16)
        h2blk = jnp.dot(h1z, hb2_s[call, :],
                        preferred_element_type=jnp.float32
                        ).astype(jnp.bfloat16)

        @pl.when(c == 0)
        def _():
            h2ab_s[rows, 0:N] = h2blk

        @pl.when(c == 1)
        def _():
            h2ab_s[rows, N:2 * N] = h2blk

    j = s - _GS

    @pl.when(jnp.logical_and(s >= _GS, s < _LS))
    def _():
        one = jnp.bfloat16(0.0)
        xb16 = x_ref[...].astype(jnp.bfloat16)
        ib = jnp.maximum(ib_ref[...], 0.0).astype(jnp.bfloat16)
        xd = xb16 * ib
        t01 = jnp.dot(xb16, h2ab_s[...],
                      preferred_element_type=jnp.float32).astype(jnp.bfloat16)
        t0 = t01[:, 0:N]
        e0 = jnp.maximum(xd, one) + jnp.maximum(t0, one)
        t1 = t01[:, N:2 * N]
        e1 = jnp.maximum(xd + jnp.maximum(t1, one), one)
        dn = (((1,), (1,)), ((), ()))
        eb0 = (jax.lax.dot_general(
            e0, linW_ref[0].astype(jnp.bfloat16), dn,
            preferred_element_type=jnp.float32)
            + linb_ref[0]).astype(jnp.bfloat16)
        eb1 = (jax.lax.dot_general(
            e1, linW_ref[1].astype(jnp.bfloat16), dn,
            preferred_element_type=jnp.float32)
            + linb_ref[1]).astype(jnp.bfloat16)
        comb = (jax.lax.dot_general(
                    eb0, projW_ref[:, 0:EMBED].astype(jnp.bfloat16), dn,
                    preferred_element_type=jnp.float32)
                + jax.lax.dot_general(
                    eb1, projW_ref[:, EMBED:2 * EMBED].astype(jnp.bfloat16),
                    dn, preferred_element_type=jnp.float32)
                + projb_ref[...]).astype(jnp.bfloat16)
        for jj in range(_NJ):
            @pl.when(j == jj)
            def _():
                for b in range(_BB):
                    xs_s[:, jj * _BB + b, :] = comb[b * T:(b + 1) * T, :]

    @pl.when(s == _LS)
    def _():
        dn = (((1,), (1,)), ((), ()))
        xs_flat = xs_s[...].reshape(T * B, EMBED)
        xg_s[...] = (jax.lax.dot_general(xs_flat,
                                         wih_ref[...].astype(jnp.bfloat16),
                                         dn,
                                         preferred_element_type=jnp.float32)
                     + bih_ref[...] + bhh_ref[...])
        tgt = sl_ref[...] - 1

        def step(t, carry):
            h, cc, acc = carry
            xt = xg_s[pl.ds(t * B, B), :]
            gates = xt + jax.lax.dot_general(
                h, whh_ref[...], dn, preferred_element_type=jnp.float32)
            i_ = jax.nn.sigmoid(gates[:, 0:RNN])
            f_ = jax.nn.sigmoid(gates[:, RNN:2 * RNN])
            g_ = jnp.tanh(gates[:, 2 * RNN:3 * RNN])
            o_ = jax.nn.sigmoid(gates[:, 3 * RNN:4 * RNN])
            cc = f_ * cc + i_ * g_
            h = o_ * jnp.tanh(cc)
            acc = jnp.where(tgt == t, h, acc)
            return h, cc, acc

        h0 = h0_ref[...]
        _, _, actual = jax.lax.fori_loop(
            0, T, step, (h0, c0_ref[...], jnp.zeros_like(h0)))
        scores = jax.lax.dot_general(actual, h2i_ref[...], dn,
                                     preferred_element_type=jnp.float32)
        probs = jax.nn.sigmoid(scores)
        ib = jnp.maximum(ib_ref[...], 0.0)
        out_ref[...] = (1.0 - ALPHA) * probs + ALPHA * (probs * ib)


def kernel(A, seqs, seq_len, h0, c0, Wc1, Wc2, Wc3, I_B, lin_W, lin_b,
           proj_W, proj_b, W_ih, W_hh, b_ih, b_hh, h2i_W):
    F = jax.nn.softmax(jnp.stack([Wc1, Wc2, Wc3]), axis=2)
    x = seqs.reshape(B * T, NB)
    ib2 = I_B.reshape(1, NB)

    out = pl.pallas_call(
        _body,
        grid=(_LS + 1,),
        in_specs=[
            pl.BlockSpec(memory_space=pltpu.SMEM),
            pl.BlockSpec((4, _RBC, N), lambda s: (0, jnp.clip(s, 0, _NIC - 1),
                                                  0)),
            pl.BlockSpec((_XB, NB), lambda s: (jnp.clip(s - _GS, 0, _NJ - 1),
                                               0)),
            pl.BlockSpec((1, NB), lambda s: (0, 0)),
            pl.BlockSpec((2, EMBED, NB), lambda s: (0, 0, 0)),
            pl.BlockSpec((2, 1, EMBED), lambda s: (0, 0, 0)),
            pl.BlockSpec((EMBED, 2 * EMBED), lambda s: (0, 0)),
            pl.BlockSpec((1, EMBED), lambda s: (0, 0)),
            pl.BlockSpec((4 * RNN, EMBED), lambda s: (0, 0)),
            pl.BlockSpec((4 * RNN, RNN), lambda s: (0, 0)),
            pl.BlockSpec((1, 4 * RNN), lambda s: (0, 0)),
            pl.BlockSpec((1, 4 * RNN), lambda s: (0, 0)),
            pl.BlockSpec((B, RNN), lambda s: (0, 0)),
            pl.BlockSpec((B, RNN), lambda s: (0, 0)),
            pl.BlockSpec((B, 1), lambda s: (0, 0)),
            pl.BlockSpec((NB, RNN), lambda s: (0, 0)),
        ],
        out_specs=pl.BlockSpec((B, NB), lambda s: (0, 0)),
        out_shape=jax.ShapeDtypeStruct((B, NB), jnp.float32),
        scratch_shapes=[
            pltpu.VMEM((2 * N, N), jnp.bfloat16),
            pltpu.VMEM((2 * N, N), jnp.bfloat16),
            pltpu.VMEM((2 * N, N), jnp.bfloat16),
            pltpu.VMEM((N, N), jnp.bfloat16),
            pltpu.VMEM((N, 2 * N), jnp.bfloat16),
            pltpu.VMEM((T, B, EMBED), jnp.bfloat16),
            pltpu.VMEM((T * B, 4 * RNN), jnp.float32),
            pltpu.VMEM((1, N), jnp.float32),
            pltpu.VMEM((1, N), jnp.float32),
        ],
        compiler_params=pltpu.CompilerParams(
            dimension_semantics=("arbitrary",)),
    )(F, A, x, ib2, lin_W, lin_b.reshape(2, 1, EMBED), proj_W,
      proj_b.reshape(1, EMBED), W_ih, W_hh, b_ih.reshape(1, 4 * RNN),
      b_hh.reshape(1, 4 * RNN), h0.reshape(B, RNN), c0.reshape(B, RNN),
      seq_len.astype(jnp.int32).reshape(B, 1), h2i_W)

    return out


# speedup vs baseline: 1.0782x; 1.0444x over previous
import jax
import jax.numpy as jnp
from jax.experimental import pallas as pl
from jax.experimental.pallas import tpu as pltpu

N = 1024
NB = 1024
EMBED = 128
RNN = 256
B = 32
T = 50
ALPHA = 0.5

_RBC = 512
_NIC = N // _RBC
_RB = 1024
_NI = N // _RB
_BB = 16
_XB = _BB * T
_NJ = B // _BB
_CS = _NIC
_GS = _CS + 4 * _NI
_LS = _GS + _NJ


def _body(F_ref, A_ref, x_ref, ib_ref, linW_ref, linb_ref, projW_ref,
          projb_ref, wih_ref, whh_ref, bih_ref, bhh_ref, h0_ref, c0_ref,
          sl_ref, h2i_ref, out_ref,
          ha_s, hb_s, hb2_s, h1z_s, h2ab_s, xs_s, xg_s, deg_s, dinv_s):
    s = pl.program_id(0)

    @pl.when(s < _CS)
    def _():
        ablk = [A_ref[e] for e in range(4)]
        for w, dst in ((0, ha_s), (1, hb_s), (2, hb2_s)):
            for cc in range(2):
                acc = F_ref[w, cc, 0] * ablk[0]
                for e in range(1, 4):
                    acc = acc + F_ref[w, cc, e] * ablk[e]
                dst[pl.ds(cc * N + s * _RBC, _RBC), :] = (
                    acc.astype(jnp.bfloat16))

    sp = s - _CS
    c = sp // (2 * _NI)
    p = (sp // _NI) % 2
    i = sp % _NI
    rows = pl.ds(i * _RB, _RB)
    crows = pl.ds(c * N + i * _RB, _RB)
    call = pl.ds(c * N, N)
    graph = jnp.logical_and(s >= _CS, s < _GS)

    @pl.when(jnp.logical_and(graph, jnp.logical_and(p == 0, i == 0)))
    def _():
        deg_s[...] = jnp.zeros_like(deg_s)

    @pl.when(jnp.logical_and(graph, p == 0))
    def _():
        h1 = jnp.dot(ha_s[crows, :], hb_s[call, :],
                     preferred_element_type=jnp.float32)
        col = jax.lax.broadcasted_iota(jnp.int32, (_RB, N), 1)
        row = jax.lax.broadcasted_iota(jnp.int32, (_RB, N), 0) + i * _RB
        h1 = jnp.where(col == row, 0.0, h1)
        deg_s[...] += jnp.sum(h1, axis=0, keepdims=True)
        h1z_s[rows, :] = h1.astype(jnp.bfloat16)

    @pl.when(jnp.logical_and(graph, jnp.logical_and(p == 1, i == 0)))
    def _():
        deg = deg_s[...]
        dinv_s[...] = jnp.where(deg > 0, 1.0 / deg, 0.0)

    @pl.when(jnp.logical_and(graph, p == 1))
    def _():
        h1z = (h1z_s[rows, :] * dinv_s[...]).astype(jnp.bfloat16)
        h2blk = jnp.dot(h1z, hb2_s[call, :],
                        preferred_element_type=jnp.float32
                        ).astype(jnp.bfloat16)

        @pl.when(c == 0)
        def _():
            h2ab_s[rows, 0:N] = h2blk

        @pl.when(c == 1)
        def _():
            h2ab_s[rows, N:2 * N] = h2blk

    j = s - _GS

    @pl.when(jnp.logical_and(s >= _GS, s < _LS))
    def _():
        one = jnp.bfloat16(0.0)
        xb16 = x_ref[...].astype(jnp.bfloat16)
        ib = jnp.maximum(ib_ref[...], 0.0).astype(jnp.bfloat16)
        xd = xb16 * ib
        t01 = jnp.dot(xb16, h2ab_s[...],
                      preferred_element_type=jnp.float32).astype(jnp.bfloat16)
        t0 = t01[:, 0:N]
        e0 = jnp.maximum(xd, one) + jnp.maximum(t0, one)
        t1 = t01[:, N:2 * N]
        e1 = jnp.maximum(xd + jnp.maximum(t1, one), one)
        dn = (((1,), (1,)), ((), ()))
        eb0 = (jax.lax.dot_general(
            e0, linW_ref[0].astype(jnp.bfloat16), dn,
            preferred_element_type=jnp.float32)
            + linb_ref[0]).astype(jnp.bfloat16)
        eb1 = (jax.lax.dot_general(
            e1, linW_ref[1].astype(jnp.bfloat16), dn,
            preferred_element_type=jnp.float32)
            + linb_ref[1]).astype(jnp.bfloat16)
        comb = (jax.lax.dot_general(
                    eb0, projW_ref[:, 0:EMBED].astype(jnp.bfloat16), dn,
                    preferred_element_type=jnp.float32)
                + jax.lax.dot_general(
                    eb1, projW_ref[:, EMBED:2 * EMBED].astype(jnp.bfloat16),
                    dn, preferred_element_type=jnp.float32)
                + projb_ref[...]).astype(jnp.bfloat16)
        for jj in range(_NJ):
            @pl.when(j == jj)
            def _():
                for b in range(_BB):
                    xs_s[:, jj * _BB + b, :] = comb[b * T:(b + 1) * T, :]

    @pl.when(s == _LS)
    def _():
        dn = (((1,), (1,)), ((), ()))
        xs_flat = xs_s[...].reshape(T * B, EMBED)
        xg_s[...] = (jax.lax.dot_general(xs_flat,
                                         wih_ref[...].astype(jnp.bfloat16),
                                         dn,
                                         preferred_element_type=jnp.float32)
                     + bih_ref[...] + bhh_ref[...])
        tgt = sl_ref[...] - 1

        def step(t, carry):
            h, cc, acc = carry
            xt = xg_s[pl.ds(t * B, B), :]
            gates = xt + jax.lax.dot_general(
                h, whh_ref[...], dn, preferred_element_type=jnp.float32)
            i_ = jax.nn.sigmoid(gates[:, 0:RNN])
            f_ = jax.nn.sigmoid(gates[:, RNN:2 * RNN])
            g_ = jnp.tanh(gates[:, 2 * RNN:3 * RNN])
            o_ = jax.nn.sigmoid(gates[:, 3 * RNN:4 * RNN])
            cc = f_ * cc + i_ * g_
            h = o_ * jnp.tanh(cc)
            acc = jnp.where(tgt == t, h, acc)
            return h, cc, acc

        h0 = h0_ref[...]
        _, _, actual = jax.lax.fori_loop(
            0, T, step, (h0, c0_ref[...], jnp.zeros_like(h0)), unroll=5)
        scores = jax.lax.dot_general(actual, h2i_ref[...], dn,
                                     preferred_element_type=jnp.float32)
        probs = jax.nn.sigmoid(scores)
        ib = jnp.maximum(ib_ref[...], 0.0)
        out_ref[...] = (1.0 - ALPHA) * probs + ALPHA * (probs * ib)


def kernel(A, seqs, seq_len, h0, c0, Wc1, Wc2, Wc3, I_B, lin_W, lin_b,
           proj_W, proj_b, W_ih, W_hh, b_ih, b_hh, h2i_W):
    F = jax.nn.softmax(jnp.stack([Wc1, Wc2, Wc3]), axis=2)
    x = seqs.reshape(B * T, NB)
    ib2 = I_B.reshape(1, NB)

    out = pl.pallas_call(
        _body,
        grid=(_LS + 1,),
        in_specs=[
            pl.BlockSpec(memory_space=pltpu.SMEM),
            pl.BlockSpec((4, _RBC, N), lambda s: (0, jnp.clip(s, 0, _NIC - 1),
                                                  0)),
            pl.BlockSpec((_XB, NB), lambda s: (jnp.clip(s - _GS, 0, _NJ - 1),
                                               0)),
            pl.BlockSpec((1, NB), lambda s: (0, 0)),
            pl.BlockSpec((2, EMBED, NB), lambda s: (0, 0, 0)),
            pl.BlockSpec((2, 1, EMBED), lambda s: (0, 0, 0)),
            pl.BlockSpec((EMBED, 2 * EMBED), lambda s: (0, 0)),
            pl.BlockSpec((1, EMBED), lambda s: (0, 0)),
            pl.BlockSpec((4 * RNN, EMBED), lambda s: (0, 0)),
            pl.BlockSpec((4 * RNN, RNN), lambda s: (0, 0)),
            pl.BlockSpec((1, 4 * RNN), lambda s: (0, 0)),
            pl.BlockSpec((1, 4 * RNN), lambda s: (0, 0)),
            pl.BlockSpec((B, RNN), lambda s: (0, 0)),
            pl.BlockSpec((B, RNN), lambda s: (0, 0)),
            pl.BlockSpec((B, 1), lambda s: (0, 0)),
            pl.BlockSpec((NB, RNN), lambda s: (0, 0)),
        ],
        out_specs=pl.BlockSpec((B, NB), lambda s: (0, 0)),
        out_shape=jax.ShapeDtypeStruct((B, NB), jnp.float32),
        scratch_shapes=[
            pltpu.VMEM((2 * N, N), jnp.bfloat16),
            pltpu.VMEM((2 * N, N), jnp.bfloat16),
            pltpu.VMEM((2 * N, N), jnp.bfloat16),
            pltpu.VMEM((N, N), jnp.bfloat16),
            pltpu.VMEM((N, 2 * N), jnp.bfloat16),
            pltpu.VMEM((T, B, EMBED), jnp.bfloat16),
            pltpu.VMEM((T * B, 4 * RNN), jnp.float32),
            pltpu.VMEM((1, N), jnp.float32),
            pltpu.VMEM((1, N), jnp.float32),
        ],
        compiler_params=pltpu.CompilerParams(
            dimension_semantics=("arbitrary",)),
    )(F, A, x, ib2, lin_W, lin_b.reshape(2, 1, EMBED), proj_W,
      proj_b.reshape(1, EMBED), W_ih, W_hh, b_ih.reshape(1, 4 * RNN),
      b_hh.reshape(1, 4 * RNN), h0.reshape(B, RNN), c0.reshape(B, RNN),
      seq_len.astype(jnp.int32).reshape(B, 1), h2i_W)

    return out


# speedup vs baseline: 1.0841x; 1.0055x over previous
import jax
import jax.numpy as jnp
from jax.experimental import pallas as pl
from jax.experimental.pallas import tpu as pltpu

N = 1024
NB = 1024
EMBED = 128
RNN = 256
B = 32
T = 50
ALPHA = 0.5

_RBC = 512
_NIC = N // _RBC
_RB = 1024
_NI = N // _RB
_BB = 16
_XB = _BB * T
_NJ = B // _BB
_CS = _NIC
_GS = _CS + 4 * _NI
_LS = _GS + _NJ


def _body(F_ref, A_ref, x_ref, ib_ref, linW_ref, linb_ref, projW_ref,
          projb_ref, wih_ref, whh_ref, bih_ref, bhh_ref, h0_ref, c0_ref,
          sl_ref, h2i_ref, out_ref,
          ha_s, hb_s, hb2_s, h1z_s, h2ab_s, xs_s, xg_s, deg_s, dinv_s):
    s = pl.program_id(0)

    @pl.when(s < _CS)
    def _():
        ablk = [A_ref[e] for e in range(4)]
        for w, dst in ((0, ha_s), (1, hb_s), (2, hb2_s)):
            for cc in range(2):
                acc = F_ref[w, cc, 0] * ablk[0]
                for e in range(1, 4):
                    acc = acc + F_ref[w, cc, e] * ablk[e]
                dst[pl.ds(cc * N + s * _RBC, _RBC), :] = (
                    acc.astype(jnp.bfloat16))

    sp = s - _CS
    c = sp // (2 * _NI)
    p = (sp // _NI) % 2
    i = sp % _NI
    rows = pl.ds(i * _RB, _RB)
    crows = pl.ds(c * N + i * _RB, _RB)
    call = pl.ds(c * N, N)
    graph = jnp.logical_and(s >= _CS, s < _GS)

    @pl.when(jnp.logical_and(graph, jnp.logical_and(p == 0, i == 0)))
    def _():
        deg_s[...] = jnp.zeros_like(deg_s)

    @pl.when(jnp.logical_and(graph, p == 0))
    def _():
        h1 = jnp.dot(ha_s[crows, :], hb_s[call, :],
                     preferred_element_type=jnp.float32)
        col = jax.lax.broadcasted_iota(jnp.int32, (_RB, N), 1)
        row = jax.lax.broadcasted_iota(jnp.int32, (_RB, N), 0) + i * _RB
        h1 = jnp.where(col == row, 0.0, h1)
        deg_s[...] += jnp.sum(h1, axis=0, keepdims=True)
        h1z_s[rows, :] = h1.astype(jnp.bfloat16)

    @pl.when(jnp.logical_and(graph, jnp.logical_and(p == 1, i == 0)))
    def _():
        deg = deg_s[...]
        dinv_s[...] = jnp.where(deg > 0, 1.0 / deg, 0.0)

    @pl.when(jnp.logical_and(graph, p == 1))
    def _():
        h1z = (h1z_s[rows, :] * dinv_s[...]).astype(jnp.bfloat16)
        h2blk = jnp.dot(h1z, hb2_s[call, :],
                        preferred_element_type=jnp.float32
                        ).astype(jnp.bfloat16)

        @pl.when(c == 0)
        def _():
            h2ab_s[rows, 0:N] = h2blk

        @pl.when(c == 1)
        def _():
            h2ab_s[rows, N:2 * N] = h2blk

    j = s - _GS

    @pl.when(jnp.logical_and(s >= _GS, s < _LS))
    def _():
        one = jnp.bfloat16(0.0)
        xb16 = x_ref[...].astype(jnp.bfloat16)
        ib = jnp.maximum(ib_ref[...], 0.0).astype(jnp.bfloat16)
        xd = xb16 * ib
        t01 = jnp.dot(xb16, h2ab_s[...],
                      preferred_element_type=jnp.float32).astype(jnp.bfloat16)
        t0 = t01[:, 0:N]
        e0 = jnp.maximum(xd, one) + jnp.maximum(t0, one)
        t1 = t01[:, N:2 * N]
        e1 = jnp.maximum(xd + jnp.maximum(t1, one), one)
        dn = (((1,), (1,)), ((), ()))
        eb0 = (jax.lax.dot_general(
            e0, linW_ref[0].astype(jnp.bfloat16), dn,
            preferred_element_type=jnp.float32)
            + linb_ref[0]).astype(jnp.bfloat16)
        eb1 = (jax.lax.dot_general(
            e1, linW_ref[1].astype(jnp.bfloat16), dn,
            preferred_element_type=jnp.float32)
            + linb_ref[1]).astype(jnp.bfloat16)
        comb = (jax.lax.dot_general(
                    eb0, projW_ref[:, 0:EMBED].astype(jnp.bfloat16), dn,
                    preferred_element_type=jnp.float32)
                + jax.lax.dot_general(
                    eb1, projW_ref[:, EMBED:2 * EMBED].astype(jnp.bfloat16),
                    dn, preferred_element_type=jnp.float32)
                + projb_ref[...]).astype(jnp.bfloat16)
        for jj in range(_NJ):
            @pl.when(j == jj)
            def _():
                for b in range(_BB):
                    xs_s[:, jj * _BB + b, :] = comb[b * T:(b + 1) * T, :]

    @pl.when(s == _LS)
    def _():
        dn = (((1,), (1,)), ((), ()))
        xs_flat = xs_s[...].reshape(T * B, EMBED)
        xg_s[...] = (jax.lax.dot_general(xs_flat,
                                         wih_ref[...].astype(jnp.bfloat16),
                                         dn,
                                         preferred_element_type=jnp.float32)
                     + bih_ref[...] + bhh_ref[...])
        tgt = sl_ref[...] - 1

        def step(t, carry):
            h, cc, acc = carry
            xt = xg_s[pl.ds(t * B, B), :]
            gates = xt + jax.lax.dot_general(
                h, whh_ref[...], dn, preferred_element_type=jnp.float32)
            i_ = jax.nn.sigmoid(gates[:, 0:RNN])
            f_ = jax.nn.sigmoid(gates[:, RNN:2 * RNN])
            g_ = jnp.tanh(gates[:, 2 * RNN:3 * RNN])
            o_ = jax.nn.sigmoid(gates[:, 3 * RNN:4 * RNN])
            cc = f_ * cc + i_ * g_
            h = o_ * jnp.tanh(cc)
            acc = jnp.where(tgt == t, h, acc)
            return h, cc, acc

        h0 = h0_ref[...]
        _, _, actual = jax.lax.fori_loop(
            0, T, step, (h0, c0_ref[...], jnp.zeros_like(h0)), unroll=10)
        scores = jax.lax.dot_general(actual, h2i_ref[...], dn,
                                     preferred_element_type=jnp.float32)
        probs = jax.nn.sigmoid(scores)
        ib = jnp.maximum(ib_ref[...], 0.0)
        out_ref[...] = (1.0 - ALPHA) * probs + ALPHA * (probs * ib)


def kernel(A, seqs, seq_len, h0, c0, Wc1, Wc2, Wc3, I_B, lin_W, lin_b,
           proj_W, proj_b, W_ih, W_hh, b_ih, b_hh, h2i_W):
    F = jax.nn.softmax(jnp.stack([Wc1, Wc2, Wc3]), axis=2)
    x = seqs.reshape(B * T, NB)
    ib2 = I_B.reshape(1, NB)

    out = pl.pallas_call(
        _body,
        grid=(_LS + 1,),
        in_specs=[
            pl.BlockSpec(memory_space=pltpu.SMEM),
            pl.BlockSpec((4, _RBC, N), lambda s: (0, jnp.clip(s, 0, _NIC - 1),
                                                  0)),
            pl.BlockSpec((_XB, NB), lambda s: (jnp.clip(s - _GS, 0, _NJ - 1),
                                               0)),
            pl.BlockSpec((1, NB), lambda s: (0, 0)),
            pl.BlockSpec((2, EMBED, NB), lambda s: (0, 0, 0)),
            pl.BlockSpec((2, 1, EMBED), lambda s: (0, 0, 0)),
            pl.BlockSpec((EMBED, 2 * EMBED), lambda s: (0, 0)),
            pl.BlockSpec((1, EMBED), lambda s: (0, 0)),
            pl.BlockSpec((4 * RNN, EMBED), lambda s: (0, 0)),
            pl.BlockSpec((4 * RNN, RNN), lambda s: (0, 0)),
            pl.BlockSpec((1, 4 * RNN), lambda s: (0, 0)),
            pl.BlockSpec((1, 4 * RNN), lambda s: (0, 0)),
            pl.BlockSpec((B, RNN), lambda s: (0, 0)),
            pl.BlockSpec((B, RNN), lambda s: (0, 0)),
            pl.BlockSpec((B, 1), lambda s: (0, 0)),
            pl.BlockSpec((NB, RNN), lambda s: (0, 0)),
        ],
        out_specs=pl.BlockSpec((B, NB), lambda s: (0, 0)),
        out_shape=jax.ShapeDtypeStruct((B, NB), jnp.float32),
        scratch_shapes=[
            pltpu.VMEM((2 * N, N), jnp.bfloat16),
            pltpu.VMEM((2 * N, N), jnp.bfloat16),
            pltpu.VMEM((2 * N, N), jnp.bfloat16),
            pltpu.VMEM((N, N), jnp.bfloat16),
            pltpu.VMEM((N, 2 * N), jnp.bfloat16),
            pltpu.VMEM((T, B, EMBED), jnp.bfloat16),
            pltpu.VMEM((T * B, 4 * RNN), jnp.float32),
            pltpu.VMEM((1, N), jnp.float32),
            pltpu.VMEM((1, N), jnp.float32),
        ],
        compiler_params=pltpu.CompilerParams(
            dimension_semantics=("arbitrary",)),
    )(F, A, x, ib2, lin_W, lin_b.reshape(2, 1, EMBED), proj_W,
      proj_b.reshape(1, EMBED), W_ih, W_hh, b_ih.reshape(1, 4 * RNN),
      b_hh.reshape(1, 4 * RNN), h0.reshape(B, RNN), c0.reshape(B, RNN),
      seq_len.astype(jnp.int32).reshape(B, 1), h2i_W)

    return out
